# baseline (speedup 1.0000x reference)
import functools

import jax
import jax.numpy as jnp
from jax import lax
from jax.experimental import pallas as pl
from jax.experimental.pallas import tpu as pltpu
from jax.experimental.pallas import tpu_sc as plsc

DIM = 128
NBUF = 14
SPLIT = 2
NC = 2
NS = 16
NW = NC * NS


@functools.lru_cache(maxsize=None)
def _make_embed(n, p):
    rows_per_w = n // NW
    sub = rows_per_w // SPLIT
    assert sub % 8 == 0 and sub <= 128
    nchunks = p * SPLIT
    laps = -(-nchunks // NBUF)

    mesh = plsc.VectorSubcoreMesh(core_axis_name="c", subcore_axis_name="s")

    @functools.partial(
        pl.kernel,
        mesh=mesh,
        out_type=jax.ShapeDtypeStruct((p, n, DIM), jnp.float32),
        scratch_types=(
            [pltpu.VMEM((p * rows_per_w,), jnp.int32)]
            + [pltpu.VMEM((sub, DIM), jnp.float32) for _ in range(NBUF)]
            + [pltpu.SemaphoreType.DMA for _ in range(2 * NBUF)]
        ),
    )
    def embed(idx_hbm, table_hbm, out_hbm, idx_v, *rest):
        bufs = rest[:NBUF]
        gsem = rest[NBUF:2 * NBUF]
        osem = rest[2 * NBUF:]
        wid = lax.axis_index("s") * NC + lax.axis_index("c")
        row0 = pl.multiple_of(wid * rows_per_w, rows_per_w)

        pltpu.sync_copy(
            idx_hbm.at[pl.ds(pl.multiple_of(wid * p * rows_per_w, 8),
                             p * rows_per_w)],
            idx_v)

        def idx_slice(c):
            return idx_v.at[pl.ds(pl.multiple_of(c * sub, 8), sub)]

        def gather_start(c, b):
            pltpu.async_copy(table_hbm.at[idx_slice(c)], bufs[b], gsem[b])

        def gather_wait(b):
            pltpu.make_async_copy(
                table_hbm.at[idx_slice(0)], bufs[b], gsem[b]).wait()

        def write_start(c, b):
            j = c // SPLIT
            h = c - j * SPLIT
            pltpu.async_copy(
                bufs[b],
                out_hbm.at[j, pl.ds(row0 + pl.multiple_of(h * sub, 8), sub)],
                osem[b])

        def write_wait(b):
            pltpu.make_async_copy(
                bufs[b], out_hbm.at[0, pl.ds(0, sub)], osem[b]).wait()

        for b in range(min(NBUF, nchunks)):
            gather_start(b, b)

        def lap(jl, carry):
            j0 = jl * NBUF
            for b in range(NBUF):
                j = j0 + b

                @pl.when(j < nchunks)
                def _():
                    gather_wait(b)
                    write_start(j, b)
            for b in range(NBUF):
                nxt = j0 + NBUF + b

                @pl.when(nxt < nchunks)
                def _():
                    write_wait(b)
                    gather_start(nxt, b)
            return carry

        lax.fori_loop(0, laps, lap, 0)
        for b in range(min(NBUF, nchunks)):
            write_wait(b)

    return embed


def kernel(input, table):
    n, p = input.shape
    idx = (input.astype(jnp.int32)
           .reshape(NW, n // NW, p)
           .transpose(0, 2, 1)
           .reshape(n * p))
    out_t = _make_embed(n, p)(idx, table.astype(jnp.float32))
    return out_t.transpose(1, 0, 2)


# speedup vs baseline: 1.0082x; 1.0082x over previous
import functools

import jax
import jax.numpy as jnp
from jax import lax
from jax.experimental import pallas as pl
from jax.experimental.pallas import tpu as pltpu
from jax.experimental.pallas import tpu_sc as plsc

DIM = 128
NBUF = 14
SPLIT = 2
NC = 2
NS = 16
NW = NC * NS


@functools.lru_cache(maxsize=None)
def _make_embed(n, p):
    rows_per_w = n // NW
    sub = rows_per_w // SPLIT
    assert sub % 8 == 0 and sub <= 128
    nchunks = p * SPLIT
    laps = -(-nchunks // NBUF)

    mesh = plsc.VectorSubcoreMesh(core_axis_name="c", subcore_axis_name="s")

    @functools.partial(
        pl.kernel,
        mesh=mesh,
        out_type=jax.ShapeDtypeStruct((p, n, DIM), jnp.float32),
        scratch_types=(
            [pltpu.VMEM((p * rows_per_w,), jnp.int32)]
            + [pltpu.VMEM((sub, DIM), jnp.float32) for _ in range(NBUF)]
            + [pltpu.SemaphoreType.DMA for _ in range(2 * NBUF)]
        ),
    )
    def embed(idx_hbm, table_hbm, out_hbm, idx_v, *rest):
        bufs = rest[:NBUF]
        gsem = rest[NBUF:2 * NBUF]
        osem = rest[2 * NBUF:]
        wid = lax.axis_index("s") * NC + lax.axis_index("c")
        row0 = pl.multiple_of(wid * rows_per_w, rows_per_w)

        pltpu.sync_copy(
            idx_hbm.at[pl.ds(pl.multiple_of(wid * p * rows_per_w, 8),
                             p * rows_per_w)],
            idx_v)

        def idx_slice(c):
            return idx_v.at[pl.ds(pl.multiple_of(c * sub, 8), sub)]

        def gather_start(c, b):
            pltpu.async_copy(table_hbm.at[idx_slice(c)], bufs[b], gsem[b])

        def gather_wait(b):
            pltpu.make_async_copy(
                table_hbm.at[idx_slice(0)], bufs[b], gsem[b]).wait()

        def write_start(c, b):
            j = c // SPLIT
            h = c - j * SPLIT
            pltpu.async_copy(
                bufs[b],
                out_hbm.at[j, pl.ds(row0 + pl.multiple_of(h * sub, 8), sub)],
                osem[b])

        def write_wait(b):
            pltpu.make_async_copy(
                bufs[b], out_hbm.at[0, pl.ds(0, sub)], osem[b]).wait()

        for b in range(min(NBUF, nchunks)):
            gather_start(b, b)

        LAG = NBUF // 2

        def refill(j0, b2):
            nxt = j0 + NBUF + b2

            @pl.when(nxt < nchunks)
            def _():
                write_wait(b2)
                gather_start(nxt, b2)

        def lap(jl, carry):
            j0 = jl * NBUF
            for b in range(NBUF):
                j = j0 + b

                @pl.when(j < nchunks)
                def _():
                    gather_wait(b)
                    write_start(j, b)
                if b >= LAG:
                    refill(j0, b - LAG)
            for b2 in range(NBUF - LAG, NBUF):
                refill(j0, b2)
            return carry

        lax.fori_loop(0, laps, lap, 0)
        for b in range(min(NBUF, nchunks)):
            write_wait(b)

    return embed


def kernel(input, table):
    n, p = input.shape
    idx = (input.astype(jnp.int32)
           .reshape(NW, n // NW, p)
           .transpose(0, 2, 1)
           .reshape(n * p))
    out_t = _make_embed(n, p)(idx, table.astype(jnp.float32))
    return out_t.transpose(1, 0, 2)


# speedup vs baseline: 1.0120x; 1.0038x over previous
import functools

import jax
import jax.numpy as jnp
from jax import lax
from jax.experimental import pallas as pl
from jax.experimental.pallas import tpu as pltpu
from jax.experimental.pallas import tpu_sc as plsc

DIM = 128
NBUF = 7
SPLIT = 1
NC = 2
NS = 16
NW = NC * NS


@functools.lru_cache(maxsize=None)
def _make_embed(n, p):
    rows_per_w = n // NW
    sub = rows_per_w // SPLIT
    assert sub % 8 == 0 and sub <= 128
    nchunks = p * SPLIT
    laps = -(-nchunks // NBUF)

    mesh = plsc.VectorSubcoreMesh(core_axis_name="c", subcore_axis_name="s")

    @functools.partial(
        pl.kernel,
        mesh=mesh,
        out_type=jax.ShapeDtypeStruct((p, n, DIM), jnp.float32),
        scratch_types=(
            [pltpu.VMEM((p * rows_per_w,), jnp.int32)]
            + [pltpu.VMEM((sub, DIM), jnp.float32) for _ in range(NBUF)]
            + [pltpu.SemaphoreType.DMA for _ in range(2 * NBUF)]
        ),
    )
    def embed(idx_hbm, table_hbm, out_hbm, idx_v, *rest):
        bufs = rest[:NBUF]
        gsem = rest[NBUF:2 * NBUF]
        osem = rest[2 * NBUF:]
        wid = lax.axis_index("s") * NC + lax.axis_index("c")
        row0 = pl.multiple_of(wid * rows_per_w, rows_per_w)

        pltpu.sync_copy(
            idx_hbm.at[pl.ds(pl.multiple_of(wid * p * rows_per_w, 8),
                             p * rows_per_w)],
            idx_v)

        def idx_slice(c):
            return idx_v.at[pl.ds(pl.multiple_of(c * sub, 8), sub)]

        def gather_start(c, b):
            pltpu.async_copy(table_hbm.at[idx_slice(c)], bufs[b], gsem[b])

        def gather_wait(b):
            pltpu.make_async_copy(
                table_hbm.at[idx_slice(0)], bufs[b], gsem[b]).wait()

        def write_start(c, b):
            j = c // SPLIT
            h = c - j * SPLIT
            pltpu.async_copy(
                bufs[b],
                out_hbm.at[j, pl.ds(row0 + pl.multiple_of(h * sub, 8), sub)],
                osem[b])

        def write_wait(b):
            pltpu.make_async_copy(
                bufs[b], out_hbm.at[0, pl.ds(0, sub)], osem[b]).wait()

        for b in range(min(NBUF, nchunks)):
            gather_start(b, b)

        LAG = NBUF // 2

        def refill(j0, b2):
            nxt = j0 + NBUF + b2

            @pl.when(nxt < nchunks)
            def _():
                write_wait(b2)
                gather_start(nxt, b2)

        def lap(jl, carry):
            j0 = jl * NBUF
            for b in range(NBUF):
                j = j0 + b

                @pl.when(j < nchunks)
                def _():
                    gather_wait(b)
                    write_start(j, b)
                if b >= LAG:
                    refill(j0, b - LAG)
            for b2 in range(NBUF - LAG, NBUF):
                refill(j0, b2)
            return carry

        lax.fori_loop(0, laps, lap, 0)
        for b in range(min(NBUF, nchunks)):
            write_wait(b)

    return embed


def kernel(input, table):
    n, p = input.shape
    idx = (input.astype(jnp.int32)
           .reshape(NW, n // NW, p)
           .transpose(0, 2, 1)
           .reshape(n * p))
    out_t = _make_embed(n, p)(idx, table.astype(jnp.float32))
    return out_t.transpose(1, 0, 2)
